# Initial kernel scaffold; baseline (speedup 1.0000x reference)
import jax
import jax.numpy as jnp
from jax.experimental import pallas as pl


def kernel(data, xyz, depth_idx, absolute_emb, depth_table):
    raise NotImplementedError("write your pallas kernel here")


# speedup vs baseline: 3661.7496x; 3661.7496x over previous
import functools
import jax
import jax.numpy as jnp
import numpy as np
from jax import lax
from jax.experimental import pallas as pl
from jax.experimental.pallas import tpu as pltpu
from jax.experimental.pallas import tpu_sc as plsc

_NUM_EMBED = 384
_N = 200000
_B = 128
_NBLK_FULL = _N // _B
_TAIL = _N - _NBLK_FULL * _B
_NBLK = _NBLK_FULL + 1
_NC, _NS, _L = 2, 16, 16
_NW = _NC * _NS
_TMAX = (_NBLK_FULL + _NW - 1) // _NW


def _body(fused_hbm, idx_hbm, out_hbm,
          shared, idx_v, ix_v, iy_v, iz_v, bufx, bufy, bufz, sem):
  cid = lax.axis_index("c")
  sid = lax.axis_index("s")
  wid = sid * _NC + cid

  @pl.when(sid == 0)
  def _stage():
    pltpu.sync_copy(fused_hbm, shared)
  plsc.subcore_barrier()

  def compute_indices():
    for g in range(_B // _L):
      sl = pl.ds(g * _L, _L)
      px = idx_v[0, sl]
      py = idx_v[1, sl]
      pz = idx_v[2, sl]
      d = idx_v[3, sl]
      ix_v[sl] = px * 4 + d
      iy_v[sl] = py * 4 + d + 512
      iz_v[sl] = pz * 4 + d + 1024

  def do_block(b, n_valid):
    pltpu.sync_copy(idx_hbm.at[b], idx_v)
    compute_indices()
    cx = pltpu.async_copy(shared.at[ix_v], bufx, sem)
    cy = pltpu.async_copy(shared.at[iy_v], bufy, sem)
    cz = pltpu.async_copy(shared.at[iz_v], bufz, sem)
    cx.wait()
    cy.wait()
    cz.wait()
    base = b * _B
    wx = pltpu.make_async_copy(
        bufx.at[pl.ds(0, n_valid)],
        out_hbm.at[pl.ds(base, n_valid), pl.ds(0, 128)], sem)
    wy = pltpu.make_async_copy(
        bufy.at[pl.ds(0, n_valid)],
        out_hbm.at[pl.ds(base, n_valid), pl.ds(128, 128)], sem)
    wz = pltpu.make_async_copy(
        bufz.at[pl.ds(0, n_valid)],
        out_hbm.at[pl.ds(base, n_valid), pl.ds(256, 128)], sem)
    wx.start()
    wy.start()
    wz.start()
    wx.wait()
    wy.wait()
    wz.wait()

  def loop_body(t, carry):
    b = wid + t * _NW

    @pl.when(b < _NBLK_FULL)
    def _():
      do_block(b, _B)
    return carry

  lax.fori_loop(0, _TMAX, loop_body, 0)

  @pl.when(wid == _NW - 1)
  def _tail():
    do_block(_NBLK_FULL, _TAIL)


@jax.jit
def _run(fused, idx_packed):
  mesh = plsc.VectorSubcoreMesh(core_axis_name="c", subcore_axis_name="s")
  return pl.kernel(
      _body,
      out_type=jax.ShapeDtypeStruct((_N, _NUM_EMBED), jnp.float32),
      mesh=mesh,
      scratch_types=[
          pltpu.VMEM_SHARED((3 * 512, 128), jnp.float32),
          pltpu.VMEM((4, _B), jnp.int32),
          pltpu.VMEM((_B,), jnp.int32),
          pltpu.VMEM((_B,), jnp.int32),
          pltpu.VMEM((_B,), jnp.int32),
          pltpu.VMEM((_B, 128), jnp.float32),
          pltpu.VMEM((_B, 128), jnp.float32),
          pltpu.VMEM((_B, 128), jnp.float32),
          pltpu.SemaphoreType.DMA,
      ],
  )(fused, idx_packed)


def kernel(data, xyz, depth_idx, absolute_emb, depth_table):
  del data
  tabs = [absolute_emb[:, a::3] for a in range(3)]
  dchunks = [depth_table[:, 128 * a:128 * (a + 1)] for a in range(3)]
  fused = jnp.concatenate(
      [(t[:, None, :] + dc[None, :, :]).reshape(512, 128)
       for t, dc in zip(tabs, dchunks)], axis=0)

  pad = _NBLK * _B - _N
  xyzp = jnp.pad(xyz, ((0, pad), (0, 0)))
  dp = jnp.pad(depth_idx, (0, pad))
  idx_packed = jnp.concatenate([xyzp, dp[:, None]], axis=1)
  idx_packed = idx_packed.reshape(_NBLK, _B, 4).transpose(0, 2, 1)

  return _run(fused, idx_packed)


# speedup vs baseline: 5720.8796x; 1.5623x over previous
import functools
import jax
import jax.numpy as jnp
import numpy as np
from jax import lax
from jax.experimental import pallas as pl
from jax.experimental.pallas import tpu as pltpu
from jax.experimental.pallas import tpu_sc as plsc

_NUM_EMBED = 384
_N = 200000
_B = 128
_NBLK_FULL = _N // _B
_NC, _NS, _L = 2, 16, 16
_NW = _NC * _NS
_T = 49
_NBLK = _NW * _T


def _body(fused_hbm, idx_hbm, out_hbm, shared, idxw,
          ix0, iy0, iz0, ix1, iy1, iz1,
          bx0, by0, bz0, bx1, by1, bz1,
          sem_i, sem_g0, sem_g1, sem_w0, sem_w1):
  cid = lax.axis_index("c")
  sid = lax.axis_index("s")
  wid = sid * _NC + cid

  ix = (ix0, ix1)
  iy = (iy0, iy1)
  iz = (iz0, iz1)
  bx = (bx0, bx1)
  by = (by0, by1)
  bz = (bz0, bz1)
  sem_g = (sem_g0, sem_g1)
  sem_w = (sem_w0, sem_w1)

  ci = pltpu.async_copy(idx_hbm.at[wid], idxw, sem_i)

  @pl.when(sid == 0)
  def _stage():
    pltpu.sync_copy(fused_hbm, shared)

  plsc.subcore_barrier()
  ci.wait()

  def out_base(t):
    b = wid + t * _NW
    base = jnp.minimum(b, _NBLK_FULL) * _B
    base = base - jnp.where(b == _NBLK_FULL, _B // 2, 0)
    return jnp.where(b > _NBLK_FULL, 0, base)

  def start_block(t, s):
    for g in range(_B // _L):
      sl = pl.ds(g * _L, _L)
      w = idxw[t, sl]
      d = lax.shift_right_logical(w, 24)
      px = w & 0xFF
      py = lax.shift_right_logical(w, 8) & 0xFF
      pz = lax.shift_right_logical(w, 16) & 0xFF
      ix[s][sl] = px * 4 + d
      iy[s][sl] = py * 4 + d + 512
      iz[s][sl] = pz * 4 + d + 1024
    pltpu.make_async_copy(shared.at[ix[s]], bx[s], sem_g[s]).start()
    pltpu.make_async_copy(shared.at[iy[s]], by[s], sem_g[s]).start()
    pltpu.make_async_copy(shared.at[iz[s]], bz[s], sem_g[s]).start()

  def wait_gathers(s):
    pltpu.make_async_copy(shared.at[ix[s]], bx[s], sem_g[s]).wait()
    pltpu.make_async_copy(shared.at[iy[s]], by[s], sem_g[s]).wait()
    pltpu.make_async_copy(shared.at[iz[s]], bz[s], sem_g[s]).wait()

  def write_descs(t, s):
    base = out_base(t)
    return (
        pltpu.make_async_copy(
            bx[s], out_hbm.at[pl.ds(base, _B), pl.ds(0, 128)], sem_w[s]),
        pltpu.make_async_copy(
            by[s], out_hbm.at[pl.ds(base, _B), pl.ds(128, 128)], sem_w[s]),
        pltpu.make_async_copy(
            bz[s], out_hbm.at[pl.ds(base, _B), pl.ds(256, 128)], sem_w[s]),
    )

  def issue_writes(t, s):
    for c in write_descs(t, s):
      c.start()

  def wait_writes(t, s):
    for c in write_descs(t, s):
      c.wait()

  start_block(0, 0)
  start_block(1, 1)
  wait_gathers(0)
  issue_writes(0, 0)
  wait_gathers(1)
  issue_writes(1, 1)

  def loop_body(i, carry):
    t0 = 2 * i
    wait_writes(t0 - 2, 0)
    start_block(t0, 0)
    wait_gathers(0)
    issue_writes(t0, 0)
    wait_writes(t0 - 1, 1)
    start_block(t0 + 1, 1)
    wait_gathers(1)
    issue_writes(t0 + 1, 1)
    return carry

  lax.fori_loop(1, (_T - 1) // 2, loop_body, 0)

  t_last = _T - 1
  wait_writes(t_last - 2, 0)
  start_block(t_last, 0)
  wait_gathers(0)
  issue_writes(t_last, 0)
  wait_writes(t_last - 1, 1)
  wait_writes(t_last, 0)


@jax.jit
def _run(fused, idx_packed):
  mesh = plsc.VectorSubcoreMesh(core_axis_name="c", subcore_axis_name="s")
  return pl.kernel(
      _body,
      out_type=jax.ShapeDtypeStruct((_N, _NUM_EMBED), jnp.float32),
      mesh=mesh,
      scratch_types=[
          pltpu.VMEM_SHARED((3 * 512, 128), jnp.float32),
          pltpu.VMEM((_T, _B), jnp.int32),
          pltpu.VMEM((_B,), jnp.int32),
          pltpu.VMEM((_B,), jnp.int32),
          pltpu.VMEM((_B,), jnp.int32),
          pltpu.VMEM((_B,), jnp.int32),
          pltpu.VMEM((_B,), jnp.int32),
          pltpu.VMEM((_B,), jnp.int32),
          pltpu.VMEM((_B, 128), jnp.float32),
          pltpu.VMEM((_B, 128), jnp.float32),
          pltpu.VMEM((_B, 128), jnp.float32),
          pltpu.VMEM((_B, 128), jnp.float32),
          pltpu.VMEM((_B, 128), jnp.float32),
          pltpu.VMEM((_B, 128), jnp.float32),
          pltpu.SemaphoreType.DMA,
          pltpu.SemaphoreType.DMA,
          pltpu.SemaphoreType.DMA,
          pltpu.SemaphoreType.DMA,
          pltpu.SemaphoreType.DMA,
      ],
  )(fused, idx_packed)


def kernel(data, xyz, depth_idx, absolute_emb, depth_table):
  del data
  tabs = [absolute_emb[:, a::3] for a in range(3)]
  dchunks = [depth_table[:, 128 * a:128 * (a + 1)] for a in range(3)]
  fused = jnp.concatenate(
      [(t[:, None, :] + dc[None, :, :]).reshape(512, 128)
       for t, dc in zip(tabs, dchunks)], axis=0)

  idxs = (xyz[:, 0] | (xyz[:, 1] << 8) | (xyz[:, 2] << 16)
          | (depth_idx << 24))
  main = idxs[:_NBLK_FULL * _B]
  tail = idxs[_N - _B:]
  n_fill = _NBLK - _NBLK_FULL - 1
  fill = jnp.tile(idxs[:_B], (n_fill,))
  blocks = jnp.concatenate([main, tail, fill], axis=0)
  blocks = blocks.reshape(_NBLK // _NW, _NW, _B)
  idx_packed = blocks.transpose(1, 0, 2)

  return _run(fused, idx_packed)
